# baseline (speedup 1.0000x reference)
import functools

import jax
import jax.numpy as jnp
import numpy as np
from jax.experimental import pallas as pl
from jax.experimental.pallas import tpu as pltpu
from jax.experimental.pallas import tpu_sc as plsc

_SIZE = 1600000
_COLS = 128
_ROWS = _SIZE // _COLS

_NC = 2
_NS = 16
_NW = _NC * _NS
_PER_W = _SIZE // _NW
_CHUNK = 10000
_NCHUNK = _PER_W // _CHUNK
_VECS = _CHUNK // 16

_NBINS = 4096
_HIST_WORDS = _NBINS * 16

_INT_MIN = np.int32(-2147483648)


def _sc_keys_hist_body(probs_hbm, noise_hbm, zeros_hbm, skey_hbm, hist_hbm,
                       pbuf, nbuf, kbuf, hist):
    wid = jax.lax.axis_index("s") * _NC + jax.lax.axis_index("c")
    lane = jax.lax.iota(jnp.int32, 16)
    laneoff = lane * _NBINS + 2048
    ones = (lane >= 0).astype(jnp.int32)

    pltpu.sync_copy(zeros_hbm, hist)

    def chunk(c, _):
        base = wid * _PER_W + c * _CHUNK
        pltpu.sync_copy(probs_hbm.at[pl.ds(base, _CHUNK)], pbuf)
        pltpu.sync_copy(noise_hbm.at[pl.ds(base, _CHUNK)], nbuf)

        def vec(i, _):
            sl = pl.ds(i * 16, 16)
            x = pbuf[sl] + nbuf[sl]
            k = plsc.bitcast(x, jnp.int32)
            s = k ^ (jax.lax.shift_right_arithmetic(k, 31)
                     & jnp.int32(0x7FFFFFFF))
            kbuf[sl] = s
            idx = laneoff + jax.lax.shift_right_arithmetic(s, 20)
            plsc.addupdate_scatter(hist, [idx], ones)
            return 0

        jax.lax.fori_loop(0, _VECS, vec, 0)
        pltpu.sync_copy(kbuf, skey_hbm.at[pl.ds(base, _CHUNK)])
        return 0

    jax.lax.fori_loop(0, _NCHUNK, chunk, 0)
    pltpu.sync_copy(hist, hist_hbm.at[wid])


def _tc_finish_body(n_ref, skey_ref, hist_ref, out_ref):
    n = n_ref[0, 0]
    total = jnp.sum(hist_ref[...], axis=0)
    r = jax.lax.broadcasted_iota(jnp.int32, (32, 128), 0)
    c = jax.lax.broadcasted_iota(jnp.int32, (32, 128), 1)
    bin2 = r * 128 + c

    def hstep(b, B):
        cand = B | jnp.left_shift(jnp.int32(1), jnp.int32(11) - b)
        cnt = jnp.sum(jnp.where(bin2 >= cand, total, 0))
        return jnp.where(cnt >= n, cand, B)

    B = jax.lax.fori_loop(0, 12, hstep, jnp.int32(0))
    cnt_above = jnp.sum(jnp.where(bin2 > B, total, 0))

    skey = skey_ref[...]
    in_bin = (jax.lax.shift_right_arithmetic(skey, 20) + 2048) == B
    low = jnp.where(in_bin, skey & jnp.int32(0xFFFFF), jnp.int32(-1))

    def lstep(b, L):
        cand = L | jnp.left_shift(jnp.int32(1), jnp.int32(19) - b)
        cnt = cnt_above + jnp.sum((low >= cand).astype(jnp.int32))
        return jnp.where(cnt >= n, cand, L)

    L = jax.lax.fori_loop(0, 20, lstep, jnp.int32(0))
    thr_s = jnp.left_shift(B ^ jnp.int32(2048), 20) | L
    out_ref[...] = (skey >= thr_s).astype(jnp.float32)


def kernel(mst_probs, n):
    noise = jax.random.gumbel(jax.random.key(42), mst_probs.shape,
                              mst_probs.dtype)
    zeros = jnp.zeros((_HIST_WORDS,), jnp.int32)

    mesh = plsc.VectorSubcoreMesh(core_axis_name="c", subcore_axis_name="s",
                                  num_cores=_NC, num_subcores=_NS)
    skeys, hists = pl.kernel(
        _sc_keys_hist_body,
        out_type=[
            jax.ShapeDtypeStruct((_SIZE,), jnp.int32),
            jax.ShapeDtypeStruct((_NW, _HIST_WORDS), jnp.int32),
        ],
        mesh=mesh,
        compiler_params=pltpu.CompilerParams(needs_layout_passes=False),
        scratch_types=[
            pltpu.VMEM((_CHUNK,), jnp.float32),
            pltpu.VMEM((_CHUNK,), jnp.float32),
            pltpu.VMEM((_CHUNK,), jnp.int32),
            pltpu.VMEM((_HIST_WORDS,), jnp.int32),
        ],
    )(mst_probs, noise, zeros)

    n_arr = jnp.asarray(n, jnp.int32).reshape(1, 1)
    out = pl.pallas_call(
        _tc_finish_body,
        out_shape=jax.ShapeDtypeStruct((_ROWS, _COLS), jnp.float32),
        in_specs=[
            pl.BlockSpec(memory_space=pltpu.SMEM),
            pl.BlockSpec(memory_space=pltpu.VMEM),
            pl.BlockSpec(memory_space=pltpu.VMEM),
        ],
        out_specs=pl.BlockSpec(memory_space=pltpu.VMEM),
    )(n_arr, skeys.reshape(_ROWS, _COLS), hists.reshape(_NW * 16, 32, 128))
    return out.reshape(_SIZE)
